# baseline (speedup 1.0000x reference)
import functools

import jax
import jax.numpy as jnp
from jax import lax
from jax.experimental import pallas as pl
from jax.experimental.pallas import tpu as pltpu
from jax.experimental.pallas import tpu_sc as plsc

_N = 10000
_E = 320000
_G = 64
_EPS = 1e-5

_EK = 128
_EP = 327680
_ER = _EP // _EK
_NT = 16
_NC = 2
_DR = _ER // _NT
_NR = _ER // (_NT * _NC)
_NP = 10240
_RT = _NP // _NT
_BN = 400
_NB = _N // _BN


def _mesh():
    return plsc.VectorSubcoreMesh(core_axis_name="c", subcore_axis_name="s")


def _rsqrt_nr(x):
    i = lax.bitcast_convert_type(x, jnp.int32)
    i = jnp.int32(0x5F3759DF) - lax.shift_right_arithmetic(i, 1)
    y = lax.bitcast_convert_type(i, jnp.float32)
    for _ in range(4):
        y = y * (1.5 - 0.5 * x * y * y)
    return y


def _norm_kernel_fn(src16, dst16, w16, norm_out,
                    sidx, wval, degv, disv, nsrc, ndst, nw, nout, sdeg):
    c = lax.axis_index("c")
    s = lax.axis_index("s")

    @pl.when(s == 0)
    def _():
        def zrow(k, _):
            disv[pl.ds(k * 16, 16)] = jnp.zeros((16,), jnp.float32)
            return 0
        lax.fori_loop(0, _N // 16, zrow, 0)
        pltpu.sync_copy(disv, sdeg)

    plsc.subcore_barrier()

    pltpu.sync_copy(src16.at[s], sidx)
    pltpu.sync_copy(w16.at[s], wval)

    def degbody(j, _):
        pltpu.sync_copy(wval.at[j], sdeg.at[sidx.at[j]], add=True)
        return 0
    lax.fori_loop(0, _DR, degbody, 0)

    plsc.subcore_barrier()

    pltpu.sync_copy(sdeg, degv)

    def disbody(k, _):
        d = degv[pl.ds(k * 16, 16)]
        y = _rsqrt_nr(jnp.maximum(d, 1e-12))
        disv[pl.ds(k * 16, 16)] = jnp.where(d > 0.0, y, 0.0)
        return 0
    lax.fori_loop(0, _N // 16, disbody, 0)

    r0 = pl.multiple_of(c * _NR, _NR)
    pltpu.sync_copy(src16.at[s, pl.ds(r0, _NR)], nsrc)
    pltpu.sync_copy(dst16.at[s, pl.ds(r0, _NR)], ndst)
    pltpu.sync_copy(w16.at[s, pl.ds(r0, _NR)], nw)

    def nbody(j, _):
        for k in range(_EK // 16):
            sl = pl.ds(k * 16, 16)
            a = plsc.load_gather(disv, [nsrc[j, sl]])
            b = plsc.load_gather(disv, [ndst[j, sl]])
            nout[j, sl] = -(a * nw[j, sl] * b)
        return 0
    lax.fori_loop(0, _NR, nbody, 0)

    pltpu.sync_copy(nout, norm_out.at[s, pl.ds(r0, _NR)])


def _make_norm_kernel():
    return functools.partial(
        pl.kernel,
        out_type=jax.ShapeDtypeStruct((_NT, _DR, _EK), jnp.float32),
        mesh=_mesh(),
        compiler_params=pltpu.CompilerParams(needs_layout_passes=False),
        scratch_types=[
            pltpu.VMEM((_DR, _EK), jnp.int32),
            pltpu.VMEM((_DR, _EK), jnp.float32),
            pltpu.VMEM((_N,), jnp.float32),
            pltpu.VMEM((_N,), jnp.float32),
            pltpu.VMEM((_NR, _EK), jnp.int32),
            pltpu.VMEM((_NR, _EK), jnp.int32),
            pltpu.VMEM((_NR, _EK), jnp.float32),
            pltpu.VMEM((_NR, _EK), jnp.float32),
            pltpu.VMEM_SHARED((_N,), jnp.float32),
        ],
    )(_norm_kernel_fn)


def _make_prop(c2):
    def prop_fn(xa, xb, src16, dst16, norm16, ya3, yb3,
                sbuf, dbuf, nbuf, gbuf0, gbuf1, zbuf, acc, sem):
        c = lax.axis_index("c")
        s = lax.axis_index("s")

        def zrow(r, _):
            for v in range(c2 // 16):
                zbuf[r, pl.ds(v * 16, 16)] = jnp.zeros((16,), jnp.float32)
            return 0
        lax.fori_loop(0, 32, zrow, 0)

        def zcopy(k, _):
            r0 = pl.multiple_of(s * _RT + k * 32, 32)
            pltpu.sync_copy(zbuf, acc.at[pl.ds(r0, 32)])
            return 0
        lax.fori_loop(0, _RT // 32, zcopy, 0)

        plsc.subcore_barrier()

        def run(x_ref):
            def gstart(j, gb):
                pltpu.make_async_copy(x_ref.at[sbuf.at[j]], gb, sem).start()

            def gwait(j, gb):
                pltpu.make_async_copy(x_ref.at[sbuf.at[j]], gb, sem).wait()

            def scale(j, gb):
                def scale16(g, _):
                    nv16 = nbuf[j, pl.ds(g * 16, 16)]
                    for l in range(16):
                        e = g * 16 + l
                        nv = jnp.full((16,), nv16[l], jnp.float32)
                        for v in range(c2 // 16):
                            sl = pl.ds(v * 16, 16)
                            gb[e, sl] = gb[e, sl] * nv
                    return 0
                lax.fori_loop(0, _EK // 16, scale16, 0)

            def super_chunk(sc_i, _):
                r0 = pl.multiple_of(sc_i * 16, 16)
                pltpu.sync_copy(src16.at[s, pl.ds(r0, 16)], sbuf)
                pltpu.sync_copy(dst16.at[s, pl.ds(r0, 16)], dbuf)
                pltpu.sync_copy(norm16.at[s, pl.ds(r0, 16)], nbuf)
                gstart(0, gbuf0)

                def pair(k, _):
                    j0 = 2 * k
                    j1 = j0 + 1
                    gwait(j0, gbuf0)
                    gstart(j1, gbuf1)
                    scale(j0, gbuf0)
                    pltpu.sync_copy(gbuf0, acc.at[dbuf.at[j0]], add=True)
                    gwait(j1, gbuf1)

                    @pl.when(k < 7)
                    def _():
                        gstart(j0 + 2, gbuf0)
                    scale(j1, gbuf1)
                    pltpu.sync_copy(gbuf1, acc.at[dbuf.at[j1]], add=True)
                    return 0
                lax.fori_loop(0, 8, pair, 0)
                return 0
            lax.fori_loop(0, _DR // 16, super_chunk, 0)

        @pl.when(c == 0)
        def _():
            run(xa)

        @pl.when(c == 1)
        def _():
            run(xb)

        plsc.subcore_barrier()

        @pl.when(c == 0)
        def _():
            pltpu.sync_copy(acc.at[pl.ds(s * _RT, _RT)], ya3.at[s])

        @pl.when(c == 1)
        def _():
            pltpu.sync_copy(acc.at[pl.ds(s * _RT, _RT)], yb3.at[s])

    return functools.partial(
        pl.kernel,
        out_type=(jax.ShapeDtypeStruct((_NT, _RT, c2), jnp.float32),
                  jax.ShapeDtypeStruct((_NT, _RT, c2), jnp.float32)),
        mesh=_mesh(),
        compiler_params=pltpu.CompilerParams(needs_layout_passes=False),
        scratch_types=[
            pltpu.VMEM((16, _EK), jnp.int32),
            pltpu.VMEM((16, _EK), jnp.int32),
            pltpu.VMEM((16, _EK), jnp.float32),
            pltpu.VMEM((_EK, c2), jnp.float32),
            pltpu.VMEM((_EK, c2), jnp.float32),
            pltpu.VMEM((32, c2), jnp.float32),
            pltpu.VMEM_SHARED((_NP, c2), jnp.float32),
            pltpu.SemaphoreType.DMA,
        ],
    )(prop_fn)


def _make_prop_es():
    c2 = 128

    def prop_fn(x, src16, dst16, norm16, p03, p13,
                sbuf, dbuf, nbuf, gbuf0, gbuf1, zbuf, acc, sem):
        c = lax.axis_index("c")
        s = lax.axis_index("s")

        def zrow(r, _):
            for v in range(c2 // 16):
                zbuf[r, pl.ds(v * 16, 16)] = jnp.zeros((16,), jnp.float32)
            return 0
        lax.fori_loop(0, 32, zrow, 0)

        def zcopy(k, _):
            r0 = pl.multiple_of(s * _RT + k * 32, 32)
            pltpu.sync_copy(zbuf, acc.at[pl.ds(r0, 32)])
            return 0
        lax.fori_loop(0, _RT // 32, zcopy, 0)

        plsc.subcore_barrier()

        cbase = c * (_DR // 2)

        def gstart(j, gb):
            pltpu.make_async_copy(x.at[sbuf.at[j]], gb, sem).start()

        def gwait(j, gb):
            pltpu.make_async_copy(x.at[sbuf.at[j]], gb, sem).wait()

        def scale(j, gb):
            def scale16(g, _):
                nv16 = nbuf[j, pl.ds(g * 16, 16)]
                for l in range(16):
                    e = g * 16 + l
                    nv = jnp.full((16,), nv16[l], jnp.float32)
                    for v in range(c2 // 16):
                        sl = pl.ds(v * 16, 16)
                        gb[e, sl] = gb[e, sl] * nv
                return 0
            lax.fori_loop(0, _EK // 16, scale16, 0)

        def super_chunk(sc_i, _):
            r0 = pl.multiple_of(cbase + sc_i * 16, 16)
            pltpu.sync_copy(src16.at[s, pl.ds(r0, 16)], sbuf)
            pltpu.sync_copy(dst16.at[s, pl.ds(r0, 16)], dbuf)
            pltpu.sync_copy(norm16.at[s, pl.ds(r0, 16)], nbuf)
            gstart(0, gbuf0)

            def pair(k, _):
                j0 = 2 * k
                j1 = j0 + 1
                gwait(j0, gbuf0)
                gstart(j1, gbuf1)
                scale(j0, gbuf0)
                pltpu.sync_copy(gbuf0, acc.at[dbuf.at[j0]], add=True)
                gwait(j1, gbuf1)

                @pl.when(k < 7)
                def _():
                    gstart(j0 + 2, gbuf0)
                scale(j1, gbuf1)
                pltpu.sync_copy(gbuf1, acc.at[dbuf.at[j1]], add=True)
                return 0
            lax.fori_loop(0, 8, pair, 0)
            return 0
        lax.fori_loop(0, _DR // 2 // 16, super_chunk, 0)

        plsc.subcore_barrier()

        @pl.when(c == 0)
        def _():
            pltpu.sync_copy(acc.at[pl.ds(s * _RT, _RT)], p03.at[s])

        @pl.when(c == 1)
        def _():
            pltpu.sync_copy(acc.at[pl.ds(s * _RT, _RT)], p13.at[s])

    return functools.partial(
        pl.kernel,
        out_type=(jax.ShapeDtypeStruct((_NT, _RT, c2), jnp.float32),
                  jax.ShapeDtypeStruct((_NT, _RT, c2), jnp.float32)),
        mesh=_mesh(),
        compiler_params=pltpu.CompilerParams(needs_layout_passes=False),
        scratch_types=[
            pltpu.VMEM((16, _EK), jnp.int32),
            pltpu.VMEM((16, _EK), jnp.int32),
            pltpu.VMEM((16, _EK), jnp.float32),
            pltpu.VMEM((_EK, c2), jnp.float32),
            pltpu.VMEM((_EK, c2), jnp.float32),
            pltpu.VMEM((32, c2), jnp.float32),
            pltpu.VMEM_SHARED((_NP, c2), jnp.float32),
            pltpu.SemaphoreType.DMA,
        ],
    )(prop_fn)


def _add_body(a_ref, b_ref, o_ref):
    o_ref[...] = a_ref[...] + b_ref[...]


def _make_add():
    return pl.pallas_call(
        _add_body,
        grid=(_NT,),
        in_specs=[
            pl.BlockSpec((_RT, 128), lambda i: (i, 0)),
            pl.BlockSpec((_RT, 128), lambda i: (i, 0)),
        ],
        out_specs=pl.BlockSpec((_RT, 128), lambda i: (i, 0)),
        out_shape=jax.ShapeDtypeStruct((_NP, 128), jnp.float32),
    )


def _pool_kernel_fn(ya, yb, batch2d, pa3, pb3, rows, bidx, macc, tmp,
                    spacc):
    c = lax.axis_index("c")
    s = lax.axis_index("s")
    gpt = _G // _NT

    def irow(r, _):
        for v in range(8):
            macc[r, pl.ds(v * 16, 16)] = jnp.full((16,), -jnp.inf, jnp.float32)
        return 0
    lax.fori_loop(0, _G, irow, 0)

    pltpu.sync_copy(batch2d.at[s], bidx.at[pl.ds(0, _RT)])

    @pl.when(c == 0)
    def _():
        pltpu.sync_copy(ya.at[s], rows)

    @pl.when(c == 1)
    def _():
        pltpu.sync_copy(yb.at[s], rows)

    def rowbody(r, _):
        g = bidx[pl.ds(r, 16)][0]
        for v in range(8):
            sl = pl.ds(v * 16, 16)
            macc[g, sl] = jnp.maximum(macc[g, sl], rows[r, sl])
        return 0
    lax.fori_loop(0, _RT, rowbody, 0)

    pltpu.sync_copy(macc, spacc.at[s])
    plsc.subcore_barrier()

    def tbody(k, _):
        pltpu.sync_copy(spacc.at[k], tmp)

        def grow(r, _):
            for v in range(8):
                sl = pl.ds(v * 16, 16)
                macc[r, sl] = jnp.maximum(macc[r, sl], tmp[r, sl])
            return 0
        lax.fori_loop(0, _G, grow, 0)
        return 0
    lax.fori_loop(0, _NT, tbody, 0)

    @pl.when(c == 0)
    def _():
        pltpu.sync_copy(macc.at[pl.ds(s * gpt, gpt)], pa3.at[s])

    @pl.when(c == 1)
    def _():
        pltpu.sync_copy(macc.at[pl.ds(s * gpt, gpt)], pb3.at[s])


def _make_pool_kernel():
    gpt = _G // _NT
    return functools.partial(
        pl.kernel,
        out_type=(jax.ShapeDtypeStruct((_NT, gpt, 128), jnp.float32),
                  jax.ShapeDtypeStruct((_NT, gpt, 128), jnp.float32)),
        mesh=_mesh(),
        compiler_params=pltpu.CompilerParams(needs_layout_passes=False),
        scratch_types=[
            pltpu.VMEM((_RT, 128), jnp.float32),
            pltpu.VMEM((_RT + 16,), jnp.int32),
            pltpu.VMEM((_G, 128), jnp.float32),
            pltpu.VMEM((_G, 128), jnp.float32),
            pltpu.VMEM_SHARED((_NT, _G, 128), jnp.float32),
        ],
    )(_pool_kernel_fn)


def _cheb_body(xa, xb, t1a, t1b, t2a, t2b, w_ref, b_ref, oh_ref,
               h_ref, s1_ref, s2_ref, cnt_ref):
    i = pl.program_id(0)
    x = jnp.concatenate([xa[...], xb[...]], axis=1)
    t1 = jnp.concatenate([t1a[...], t1b[...]], axis=1)
    t2 = jnp.concatenate([t2a[...], t2b[...]], axis=1)
    a0 = w_ref[0] - w_ref[2]
    a1 = w_ref[1]
    a2 = 2.0 * w_ref[2]
    h = (jnp.dot(x, a0, preferred_element_type=jnp.float32)
         + jnp.dot(t1, a1, preferred_element_type=jnp.float32)
         + jnp.dot(t2, a2, preferred_element_type=jnp.float32)
         + b_ref[...])
    h_ref[...] = h
    oh = oh_ref[...]

    @pl.when(i == 0)
    def _():
        s1_ref[...] = jnp.zeros_like(s1_ref)
        s2_ref[...] = jnp.zeros_like(s2_ref)
        cnt_ref[...] = jnp.zeros_like(cnt_ref)

    dn = (((0,), (0,)), ((), ()))
    s1_ref[...] += lax.dot_general(oh, h, dn, preferred_element_type=jnp.float32)
    s2_ref[...] += lax.dot_general(oh, h * h, dn,
                                   preferred_element_type=jnp.float32)
    cnt_ref[...] += lax.dot_general(oh, jnp.ones((_BN, 128), jnp.float32), dn,
                                    preferred_element_type=jnp.float32)


def _make_cheb(cin):
    c2 = cin // 2
    return pl.pallas_call(
        _cheb_body,
        grid=(_NB,),
        in_specs=[
            pl.BlockSpec((_BN, c2), lambda i: (i, 0)),
            pl.BlockSpec((_BN, c2), lambda i: (i, 0)),
            pl.BlockSpec((_BN, c2), lambda i: (i, 0)),
            pl.BlockSpec((_BN, c2), lambda i: (i, 0)),
            pl.BlockSpec((_BN, c2), lambda i: (i, 0)),
            pl.BlockSpec((_BN, c2), lambda i: (i, 0)),
            pl.BlockSpec((3, cin, 256), lambda i: (0, 0, 0)),
            pl.BlockSpec((1, 256), lambda i: (0, 0)),
            pl.BlockSpec((_BN, _G), lambda i: (i, 0)),
        ],
        out_specs=[
            pl.BlockSpec((_BN, 256), lambda i: (i, 0)),
            pl.BlockSpec((_G, 256), lambda i: (0, 0)),
            pl.BlockSpec((_G, 256), lambda i: (0, 0)),
            pl.BlockSpec((_G, 128), lambda i: (0, 0)),
        ],
        out_shape=[
            jax.ShapeDtypeStruct((_N, 256), jnp.float32),
            jax.ShapeDtypeStruct((_G, 256), jnp.float32),
            jax.ShapeDtypeStruct((_G, 256), jnp.float32),
            jax.ShapeDtypeStruct((_G, 128), jnp.float32),
        ],
    )


def _cheb1_body(x_ref, t1_ref, q0_ref, q1_ref, w_ref, b_ref, oh_ref,
                h_ref, s1_ref, s2_ref, cnt_ref):
    i = pl.program_id(0)
    x = x_ref[...]
    t1 = t1_ref[...]
    t2 = q0_ref[...] + q1_ref[...]
    a0 = w_ref[0] - w_ref[2]
    a1 = w_ref[1]
    a2 = 2.0 * w_ref[2]
    h = (jnp.dot(x, a0, preferred_element_type=jnp.float32)
         + jnp.dot(t1, a1, preferred_element_type=jnp.float32)
         + jnp.dot(t2, a2, preferred_element_type=jnp.float32)
         + b_ref[...])
    h_ref[...] = h
    oh = oh_ref[...]

    @pl.when(i == 0)
    def _():
        s1_ref[...] = jnp.zeros_like(s1_ref)
        s2_ref[...] = jnp.zeros_like(s2_ref)
        cnt_ref[...] = jnp.zeros_like(cnt_ref)

    dn = (((0,), (0,)), ((), ()))
    s1_ref[...] += lax.dot_general(oh, h, dn, preferred_element_type=jnp.float32)
    s2_ref[...] += lax.dot_general(oh, h * h, dn,
                                   preferred_element_type=jnp.float32)
    cnt_ref[...] += lax.dot_general(oh, jnp.ones((_BN, 128), jnp.float32), dn,
                                    preferred_element_type=jnp.float32)


def _make_cheb1():
    return pl.pallas_call(
        _cheb1_body,
        grid=(_NB,),
        in_specs=[
            pl.BlockSpec((_BN, 128), lambda i: (i, 0)),
            pl.BlockSpec((_BN, 128), lambda i: (i, 0)),
            pl.BlockSpec((_BN, 128), lambda i: (i, 0)),
            pl.BlockSpec((_BN, 128), lambda i: (i, 0)),
            pl.BlockSpec((3, 128, 256), lambda i: (0, 0, 0)),
            pl.BlockSpec((1, 256), lambda i: (0, 0)),
            pl.BlockSpec((_BN, _G), lambda i: (i, 0)),
        ],
        out_specs=[
            pl.BlockSpec((_BN, 256), lambda i: (i, 0)),
            pl.BlockSpec((_G, 256), lambda i: (0, 0)),
            pl.BlockSpec((_G, 256), lambda i: (0, 0)),
            pl.BlockSpec((_G, 128), lambda i: (0, 0)),
        ],
        out_shape=[
            jax.ShapeDtypeStruct((_N, 256), jnp.float32),
            jax.ShapeDtypeStruct((_G, 256), jnp.float32),
            jax.ShapeDtypeStruct((_G, 256), jnp.float32),
            jax.ShapeDtypeStruct((_G, 128), jnp.float32),
        ],
    )


def _gn_body(h_ref, oh_ref, s1_ref, s2_ref, cnt_ref, gw_ref, gb_ref, gms_ref,
             ya_ref, yb_ref):
    cnt = jnp.maximum(cnt_ref[...][:, 0:1], 1.0)
    mean = s1_ref[...] / cnt
    msq = s2_ref[...] / cnt
    ms = gms_ref[...]
    var = msq - mean * mean * (ms * (2.0 - ms))
    rstd = lax.rsqrt(var + _EPS)
    oh = oh_ref[...]
    meanb = jnp.dot(oh, mean * ms, preferred_element_type=jnp.float32)
    rstdb = jnp.dot(oh, rstd, preferred_element_type=jnp.float32)
    y = jnp.maximum((h_ref[...] - meanb) * rstdb * gw_ref[...] + gb_ref[...],
                    0.0)
    ya_ref[...] = y[:, :128]
    yb_ref[...] = y[:, 128:]


def _make_gn():
    return pl.pallas_call(
        _gn_body,
        grid=(_NB,),
        in_specs=[
            pl.BlockSpec((_BN, 256), lambda i: (i, 0)),
            pl.BlockSpec((_BN, _G), lambda i: (i, 0)),
            pl.BlockSpec((_G, 256), lambda i: (0, 0)),
            pl.BlockSpec((_G, 256), lambda i: (0, 0)),
            pl.BlockSpec((_G, 128), lambda i: (0, 0)),
            pl.BlockSpec((1, 256), lambda i: (0, 0)),
            pl.BlockSpec((1, 256), lambda i: (0, 0)),
            pl.BlockSpec((1, 256), lambda i: (0, 0)),
        ],
        out_specs=[
            pl.BlockSpec((_BN, 128), lambda i: (i, 0)),
            pl.BlockSpec((_BN, 128), lambda i: (i, 0)),
        ],
        out_shape=[
            jax.ShapeDtypeStruct((_N, 128), jnp.float32),
            jax.ShapeDtypeStruct((_N, 128), jnp.float32),
        ],
    )


def _lin_body(pa_ref, pb_ref, w_ref, b_ref, out_ref):
    p = jnp.concatenate([pa_ref[...], pb_ref[...]], axis=1)
    p = jnp.where(jnp.isfinite(p), p, 0.0)
    out_ref[...] = jnp.dot(p, w_ref[...],
                           preferred_element_type=jnp.float32) + b_ref[...]


def _make_lin():
    return pl.pallas_call(
        _lin_body,
        out_shape=jax.ShapeDtypeStruct((_G, 128), jnp.float32),
    )


def _pad_nodes(a):
    return jnp.pad(a, ((0, _NP - _N), (0, 0)))


def kernel(x, edge_index, edge_weight, batch,
           W1, b1, gn1_w, gn1_b, gn1_ms,
           W2, b2, gn2_w, gn2_b, gn2_ms,
           W3, b3, gn3_w, gn3_b, gn3_ms,
           lin_w, lin_b):
    npad = _EP - _E
    pad_idx = (jnp.arange(npad, dtype=jnp.int32) * 7) % _N
    src_p = jnp.concatenate([edge_index[0], pad_idx])
    dst_p = jnp.concatenate([edge_index[1], pad_idx])
    w_p = jnp.concatenate([edge_weight, jnp.zeros((npad,), jnp.float32)])

    src16 = src_p.reshape(_NT, _DR, _EK)
    dst16 = dst_p.reshape(_NT, _DR, _EK)
    w16 = w_p.reshape(_NT, _DR, _EK)

    norm16 = _make_norm_kernel()(src16, dst16, w16)

    oh = (batch[:, None] == jnp.arange(_G, dtype=batch.dtype)[None, :]
          ).astype(jnp.float32)
    batch_p = jnp.concatenate(
        [batch, jnp.full((_NP - _N,), _G - 1, jnp.int32)]).reshape(_NT, _RT)

    prop = _make_prop(128)
    prop_es = _make_prop_es()
    addk = _make_add()
    cheb1 = _make_cheb1()
    cheb256 = _make_cheb(256)
    gn = _make_gn()

    def layer(xa, xb, W, b, gw, gb, gms):
        xap = _pad_nodes(xa)
        xbp = _pad_nodes(xb)
        t1a3, t1b3 = prop(xap, xbp, src16, dst16, norm16)
        t1ap = t1a3.reshape(_NP, 128)
        t1bp = t1b3.reshape(_NP, 128)
        t2a3, t2b3 = prop(t1ap, t1bp, src16, dst16, norm16)
        h, s1, s2, cnt = cheb256(xa, xb, t1ap, t1bp,
                                 t2a3.reshape(_NP, 128),
                                 t2b3.reshape(_NP, 128), W,
                                 b.reshape(1, 256), oh)
        ya, yb = gn(h, oh, s1, s2, cnt, gw.reshape(1, 256),
                    gb.reshape(1, 256), gms.reshape(1, 256))
        return ya, yb

    xp = _pad_nodes(x)
    p03, p13 = prop_es(xp, src16, dst16, norm16)
    t1p = addk(p03.reshape(_NP, 128), p13.reshape(_NP, 128))
    q03, q13 = prop_es(t1p, src16, dst16, norm16)
    h1, s11, s21, cnt1 = cheb1(x, t1p, q03.reshape(_NP, 128),
                               q13.reshape(_NP, 128), W1,
                               b1.reshape(1, 256), oh)
    y1a, y1b = gn(h1, oh, s11, s21, cnt1, gn1_w.reshape(1, 256),
                  gn1_b.reshape(1, 256), gn1_ms.reshape(1, 256))

    y2a, y2b = layer(y1a, y1b, W2, b2, gn2_w, gn2_b, gn2_ms)
    y3a, y3b = layer(y2a, y2b, W3, b3, gn3_w, gn3_b, gn3_ms)

    pa3, pb3 = _make_pool_kernel()(
        _pad_nodes(y3a).reshape(_NT, _RT, 128),
        _pad_nodes(y3b).reshape(_NT, _RT, 128), batch_p)

    lw = jnp.pad(lin_w, ((0, 0), (0, 112)))
    lb = jnp.pad(lin_b, (0, 112)).reshape(1, 128)
    out = _make_lin()(pa3.reshape(_G, 128), pb3.reshape(_G, 128), lw, lb)
    return out[:, :16]


# speedup vs baseline: 1.0299x; 1.0299x over previous
import functools

import jax
import jax.numpy as jnp
from jax import lax
from jax.experimental import pallas as pl
from jax.experimental.pallas import tpu as pltpu
from jax.experimental.pallas import tpu_sc as plsc

_N = 10000
_E = 320000
_G = 64
_EPS = 1e-5

_EK = 128
_EP = 327680
_ER = _EP // _EK
_NT = 16
_NC = 2
_DR = _ER // _NT
_NR = _ER // (_NT * _NC)
_NP = 10240
_RT = _NP // _NT
_BN = 400
_NB = _N // _BN


def _mesh():
    return plsc.VectorSubcoreMesh(core_axis_name="c", subcore_axis_name="s")


def _rsqrt_nr(x):
    i = lax.bitcast_convert_type(x, jnp.int32)
    i = jnp.int32(0x5F3759DF) - lax.shift_right_arithmetic(i, 1)
    y = lax.bitcast_convert_type(i, jnp.float32)
    for _ in range(4):
        y = y * (1.5 - 0.5 * x * y * y)
    return y


def _norm_kernel_fn(src16, dst16, w16, norm_out,
                    sidx, wval, degv, disv, nsrc, ndst, nw, nout, sdeg):
    c = lax.axis_index("c")
    s = lax.axis_index("s")

    @pl.when(s == 0)
    def _():
        def zrow(k, _):
            disv[pl.ds(k * 16, 16)] = jnp.zeros((16,), jnp.float32)
            return 0
        lax.fori_loop(0, _N // 16, zrow, 0)
        pltpu.sync_copy(disv, sdeg)

    plsc.subcore_barrier()

    pltpu.sync_copy(src16.at[s], sidx)
    pltpu.sync_copy(w16.at[s], wval)

    def degbody(j, _):
        pltpu.sync_copy(wval.at[j], sdeg.at[sidx.at[j]], add=True)
        return 0
    lax.fori_loop(0, _DR, degbody, 0)

    plsc.subcore_barrier()

    pltpu.sync_copy(sdeg, degv)

    def disbody(k, _):
        d = degv[pl.ds(k * 16, 16)]
        y = _rsqrt_nr(jnp.maximum(d, 1e-12))
        disv[pl.ds(k * 16, 16)] = jnp.where(d > 0.0, y, 0.0)
        return 0
    lax.fori_loop(0, _N // 16, disbody, 0)

    r0 = pl.multiple_of(c * _NR, _NR)
    pltpu.sync_copy(src16.at[s, pl.ds(r0, _NR)], nsrc)
    pltpu.sync_copy(dst16.at[s, pl.ds(r0, _NR)], ndst)
    pltpu.sync_copy(w16.at[s, pl.ds(r0, _NR)], nw)

    def nbody(j, _):
        for k in range(_EK // 16):
            sl = pl.ds(k * 16, 16)
            a = plsc.load_gather(disv, [nsrc[j, sl]])
            b = plsc.load_gather(disv, [ndst[j, sl]])
            nout[j, sl] = -(a * nw[j, sl] * b)
        return 0
    lax.fori_loop(0, _NR, nbody, 0)

    pltpu.sync_copy(nout, norm_out.at[s, pl.ds(r0, _NR)])


def _make_norm_kernel():
    return functools.partial(
        pl.kernel,
        out_type=jax.ShapeDtypeStruct((_NT, _DR, _EK), jnp.float32),
        mesh=_mesh(),
        compiler_params=pltpu.CompilerParams(needs_layout_passes=False),
        scratch_types=[
            pltpu.VMEM((_DR, _EK), jnp.int32),
            pltpu.VMEM((_DR, _EK), jnp.float32),
            pltpu.VMEM((_N,), jnp.float32),
            pltpu.VMEM((_N,), jnp.float32),
            pltpu.VMEM((_NR, _EK), jnp.int32),
            pltpu.VMEM((_NR, _EK), jnp.int32),
            pltpu.VMEM((_NR, _EK), jnp.float32),
            pltpu.VMEM((_NR, _EK), jnp.float32),
            pltpu.VMEM_SHARED((_N,), jnp.float32),
        ],
    )(_norm_kernel_fn)


def _make_prop(c2):
    def prop_fn(xa, xb, src16, dst16, norm16, ya3, yb3,
                sbuf, dbuf, nbuf, gbuf0, gbuf1, zbuf, acc, sem):
        c = lax.axis_index("c")
        s = lax.axis_index("s")

        def zrow(r, _):
            for v in range(c2 // 16):
                zbuf[r, pl.ds(v * 16, 16)] = jnp.zeros((16,), jnp.float32)
            return 0
        lax.fori_loop(0, 16, zrow, 0)

        def zcopy(k, _):
            r0 = pl.multiple_of(s * _RT + k * 16, 16)
            pltpu.sync_copy(zbuf, acc.at[pl.ds(r0, 16)])
            return 0
        lax.fori_loop(0, _RT // 16, zcopy, 0)

        plsc.subcore_barrier()

        def run(x_ref):
            def gstart(j, gb):
                pltpu.make_async_copy(x_ref.at[sbuf.at[j]], gb, sem).start()

            def gwait(j, gb):
                pltpu.make_async_copy(x_ref.at[sbuf.at[j]], gb, sem).wait()

            def scale(j, gb):
                def scale16(g, _):
                    nv16 = nbuf[j, pl.ds(g * 16, 16)]
                    for l in range(16):
                        e = g * 16 + l
                        nv = jnp.full((16,), nv16[l], jnp.float32)
                        for v in range(c2 // 16):
                            sl = pl.ds(v * 16, 16)
                            gb[e, sl] = gb[e, sl] * nv
                    return 0
                lax.fori_loop(0, _EK // 16, scale16, 0)

            def super_chunk(sc_i, _):
                r0 = pl.multiple_of(sc_i * 32, 32)
                pltpu.sync_copy(src16.at[s, pl.ds(r0, 32)], sbuf)
                pltpu.sync_copy(dst16.at[s, pl.ds(r0, 32)], dbuf)
                pltpu.sync_copy(norm16.at[s, pl.ds(r0, 32)], nbuf)
                gstart(0, gbuf0)

                def pair(k, _):
                    j0 = 2 * k
                    j1 = j0 + 1
                    gwait(j0, gbuf0)
                    gstart(j1, gbuf1)
                    scale(j0, gbuf0)
                    pltpu.sync_copy(gbuf0, acc.at[dbuf.at[j0]], add=True)
                    gwait(j1, gbuf1)

                    @pl.when(k < 15)
                    def _():
                        gstart(j0 + 2, gbuf0)
                    scale(j1, gbuf1)
                    pltpu.sync_copy(gbuf1, acc.at[dbuf.at[j1]], add=True)
                    return 0
                lax.fori_loop(0, 16, pair, 0)
                return 0
            lax.fori_loop(0, _DR // 32, super_chunk, 0)

        @pl.when(c == 0)
        def _():
            run(xa)

        @pl.when(c == 1)
        def _():
            run(xb)

        plsc.subcore_barrier()

        @pl.when(c == 0)
        def _():
            pltpu.sync_copy(acc.at[pl.ds(s * _RT, _RT)], ya3.at[s])

        @pl.when(c == 1)
        def _():
            pltpu.sync_copy(acc.at[pl.ds(s * _RT, _RT)], yb3.at[s])

    return functools.partial(
        pl.kernel,
        out_type=(jax.ShapeDtypeStruct((_NT, _RT, c2), jnp.float32),
                  jax.ShapeDtypeStruct((_NT, _RT, c2), jnp.float32)),
        mesh=_mesh(),
        compiler_params=pltpu.CompilerParams(needs_layout_passes=False),
        scratch_types=[
            pltpu.VMEM((32, _EK), jnp.int32),
            pltpu.VMEM((32, _EK), jnp.int32),
            pltpu.VMEM((32, _EK), jnp.float32),
            pltpu.VMEM((_EK, c2), jnp.float32),
            pltpu.VMEM((_EK, c2), jnp.float32),
            pltpu.VMEM((16, c2), jnp.float32),
            pltpu.VMEM_SHARED((_NP, c2), jnp.float32),
            pltpu.SemaphoreType.DMA,
        ],
    )(prop_fn)


def _make_prop_es():
    c2 = 128

    def prop_fn(x, src16, dst16, norm16, p03, p13,
                sbuf, dbuf, nbuf, gbuf0, gbuf1, zbuf, acc, sem):
        c = lax.axis_index("c")
        s = lax.axis_index("s")

        def zrow(r, _):
            for v in range(c2 // 16):
                zbuf[r, pl.ds(v * 16, 16)] = jnp.zeros((16,), jnp.float32)
            return 0
        lax.fori_loop(0, 32, zrow, 0)

        def zcopy(k, _):
            r0 = pl.multiple_of(s * _RT + k * 32, 32)
            pltpu.sync_copy(zbuf, acc.at[pl.ds(r0, 32)])
            return 0
        lax.fori_loop(0, _RT // 32, zcopy, 0)

        plsc.subcore_barrier()

        cbase = c * (_DR // 2)

        def gstart(j, gb):
            pltpu.make_async_copy(x.at[sbuf.at[j]], gb, sem).start()

        def gwait(j, gb):
            pltpu.make_async_copy(x.at[sbuf.at[j]], gb, sem).wait()

        def scale(j, gb):
            def scale16(g, _):
                nv16 = nbuf[j, pl.ds(g * 16, 16)]
                for l in range(16):
                    e = g * 16 + l
                    nv = jnp.full((16,), nv16[l], jnp.float32)
                    for v in range(c2 // 16):
                        sl = pl.ds(v * 16, 16)
                        gb[e, sl] = gb[e, sl] * nv
                return 0
            lax.fori_loop(0, _EK // 16, scale16, 0)

        def super_chunk(sc_i, _):
            r0 = pl.multiple_of(cbase + sc_i * 16, 16)
            pltpu.sync_copy(src16.at[s, pl.ds(r0, 16)], sbuf)
            pltpu.sync_copy(dst16.at[s, pl.ds(r0, 16)], dbuf)
            pltpu.sync_copy(norm16.at[s, pl.ds(r0, 16)], nbuf)
            gstart(0, gbuf0)

            def pair(k, _):
                j0 = 2 * k
                j1 = j0 + 1
                gwait(j0, gbuf0)
                gstart(j1, gbuf1)
                scale(j0, gbuf0)
                pltpu.sync_copy(gbuf0, acc.at[dbuf.at[j0]], add=True)
                gwait(j1, gbuf1)

                @pl.when(k < 7)
                def _():
                    gstart(j0 + 2, gbuf0)
                scale(j1, gbuf1)
                pltpu.sync_copy(gbuf1, acc.at[dbuf.at[j1]], add=True)
                return 0
            lax.fori_loop(0, 8, pair, 0)
            return 0
        lax.fori_loop(0, _DR // 2 // 16, super_chunk, 0)

        plsc.subcore_barrier()

        @pl.when(c == 0)
        def _():
            pltpu.sync_copy(acc.at[pl.ds(s * _RT, _RT)], p03.at[s])

        @pl.when(c == 1)
        def _():
            pltpu.sync_copy(acc.at[pl.ds(s * _RT, _RT)], p13.at[s])

    return functools.partial(
        pl.kernel,
        out_type=(jax.ShapeDtypeStruct((_NT, _RT, c2), jnp.float32),
                  jax.ShapeDtypeStruct((_NT, _RT, c2), jnp.float32)),
        mesh=_mesh(),
        compiler_params=pltpu.CompilerParams(needs_layout_passes=False),
        scratch_types=[
            pltpu.VMEM((16, _EK), jnp.int32),
            pltpu.VMEM((16, _EK), jnp.int32),
            pltpu.VMEM((16, _EK), jnp.float32),
            pltpu.VMEM((_EK, c2), jnp.float32),
            pltpu.VMEM((_EK, c2), jnp.float32),
            pltpu.VMEM((32, c2), jnp.float32),
            pltpu.VMEM_SHARED((_NP, c2), jnp.float32),
            pltpu.SemaphoreType.DMA,
        ],
    )(prop_fn)


def _add_body(a_ref, b_ref, o_ref):
    o_ref[...] = a_ref[...] + b_ref[...]


def _make_add():
    return pl.pallas_call(
        _add_body,
        grid=(_NT,),
        in_specs=[
            pl.BlockSpec((_RT, 128), lambda i: (i, 0)),
            pl.BlockSpec((_RT, 128), lambda i: (i, 0)),
        ],
        out_specs=pl.BlockSpec((_RT, 128), lambda i: (i, 0)),
        out_shape=jax.ShapeDtypeStruct((_NP, 128), jnp.float32),
    )


def _pool_kernel_fn(ya, yb, batch2d, pa3, pb3, rows, bidx, macc, tmp,
                    spacc):
    c = lax.axis_index("c")
    s = lax.axis_index("s")
    gpt = _G // _NT

    def irow(r, _):
        for v in range(8):
            macc[r, pl.ds(v * 16, 16)] = jnp.full((16,), -jnp.inf, jnp.float32)
        return 0
    lax.fori_loop(0, _G, irow, 0)

    pltpu.sync_copy(batch2d.at[s], bidx.at[pl.ds(0, _RT)])

    @pl.when(c == 0)
    def _():
        pltpu.sync_copy(ya.at[s], rows)

    @pl.when(c == 1)
    def _():
        pltpu.sync_copy(yb.at[s], rows)

    def rowbody(r, _):
        g = bidx[pl.ds(r, 16)][0]
        for v in range(8):
            sl = pl.ds(v * 16, 16)
            macc[g, sl] = jnp.maximum(macc[g, sl], rows[r, sl])
        return 0
    lax.fori_loop(0, _RT, rowbody, 0)

    pltpu.sync_copy(macc, spacc.at[s])
    plsc.subcore_barrier()

    def tbody(k, _):
        pltpu.sync_copy(spacc.at[k], tmp)

        def grow(r, _):
            for v in range(8):
                sl = pl.ds(v * 16, 16)
                macc[r, sl] = jnp.maximum(macc[r, sl], tmp[r, sl])
            return 0
        lax.fori_loop(0, _G, grow, 0)
        return 0
    lax.fori_loop(0, _NT, tbody, 0)

    @pl.when(c == 0)
    def _():
        pltpu.sync_copy(macc.at[pl.ds(s * gpt, gpt)], pa3.at[s])

    @pl.when(c == 1)
    def _():
        pltpu.sync_copy(macc.at[pl.ds(s * gpt, gpt)], pb3.at[s])


def _make_pool_kernel():
    gpt = _G // _NT
    return functools.partial(
        pl.kernel,
        out_type=(jax.ShapeDtypeStruct((_NT, gpt, 128), jnp.float32),
                  jax.ShapeDtypeStruct((_NT, gpt, 128), jnp.float32)),
        mesh=_mesh(),
        compiler_params=pltpu.CompilerParams(needs_layout_passes=False),
        scratch_types=[
            pltpu.VMEM((_RT, 128), jnp.float32),
            pltpu.VMEM((_RT + 16,), jnp.int32),
            pltpu.VMEM((_G, 128), jnp.float32),
            pltpu.VMEM((_G, 128), jnp.float32),
            pltpu.VMEM_SHARED((_NT, _G, 128), jnp.float32),
        ],
    )(_pool_kernel_fn)


def _cheb_body(xa, xb, t1a, t1b, t2a, t2b, w_ref, b_ref, oh_ref,
               h_ref, s1_ref, s2_ref, cnt_ref):
    i = pl.program_id(0)
    x = jnp.concatenate([xa[...], xb[...]], axis=1)
    t1 = jnp.concatenate([t1a[...], t1b[...]], axis=1)
    t2 = jnp.concatenate([t2a[...], t2b[...]], axis=1)
    a0 = w_ref[0] - w_ref[2]
    a1 = w_ref[1]
    a2 = 2.0 * w_ref[2]
    h = (jnp.dot(x, a0, preferred_element_type=jnp.float32)
         + jnp.dot(t1, a1, preferred_element_type=jnp.float32)
         + jnp.dot(t2, a2, preferred_element_type=jnp.float32)
         + b_ref[...])
    h_ref[...] = h
    oh = oh_ref[...]

    @pl.when(i == 0)
    def _():
        s1_ref[...] = jnp.zeros_like(s1_ref)
        s2_ref[...] = jnp.zeros_like(s2_ref)
        cnt_ref[...] = jnp.zeros_like(cnt_ref)

    dn = (((0,), (0,)), ((), ()))
    s1_ref[...] += lax.dot_general(oh, h, dn, preferred_element_type=jnp.float32)
    s2_ref[...] += lax.dot_general(oh, h * h, dn,
                                   preferred_element_type=jnp.float32)
    cnt_ref[...] += lax.dot_general(oh, jnp.ones((_BN, 128), jnp.float32), dn,
                                    preferred_element_type=jnp.float32)


def _make_cheb(cin):
    c2 = cin // 2
    return pl.pallas_call(
        _cheb_body,
        grid=(_NB,),
        in_specs=[
            pl.BlockSpec((_BN, c2), lambda i: (i, 0)),
            pl.BlockSpec((_BN, c2), lambda i: (i, 0)),
            pl.BlockSpec((_BN, c2), lambda i: (i, 0)),
            pl.BlockSpec((_BN, c2), lambda i: (i, 0)),
            pl.BlockSpec((_BN, c2), lambda i: (i, 0)),
            pl.BlockSpec((_BN, c2), lambda i: (i, 0)),
            pl.BlockSpec((3, cin, 256), lambda i: (0, 0, 0)),
            pl.BlockSpec((1, 256), lambda i: (0, 0)),
            pl.BlockSpec((_BN, _G), lambda i: (i, 0)),
        ],
        out_specs=[
            pl.BlockSpec((_BN, 256), lambda i: (i, 0)),
            pl.BlockSpec((_G, 256), lambda i: (0, 0)),
            pl.BlockSpec((_G, 256), lambda i: (0, 0)),
            pl.BlockSpec((_G, 128), lambda i: (0, 0)),
        ],
        out_shape=[
            jax.ShapeDtypeStruct((_N, 256), jnp.float32),
            jax.ShapeDtypeStruct((_G, 256), jnp.float32),
            jax.ShapeDtypeStruct((_G, 256), jnp.float32),
            jax.ShapeDtypeStruct((_G, 128), jnp.float32),
        ],
    )


def _cheb1_body(x_ref, t1_ref, q0_ref, q1_ref, w_ref, b_ref, oh_ref,
                h_ref, s1_ref, s2_ref, cnt_ref):
    i = pl.program_id(0)
    x = x_ref[...]
    t1 = t1_ref[...]
    t2 = q0_ref[...] + q1_ref[...]
    a0 = w_ref[0] - w_ref[2]
    a1 = w_ref[1]
    a2 = 2.0 * w_ref[2]
    h = (jnp.dot(x, a0, preferred_element_type=jnp.float32)
         + jnp.dot(t1, a1, preferred_element_type=jnp.float32)
         + jnp.dot(t2, a2, preferred_element_type=jnp.float32)
         + b_ref[...])
    h_ref[...] = h
    oh = oh_ref[...]

    @pl.when(i == 0)
    def _():
        s1_ref[...] = jnp.zeros_like(s1_ref)
        s2_ref[...] = jnp.zeros_like(s2_ref)
        cnt_ref[...] = jnp.zeros_like(cnt_ref)

    dn = (((0,), (0,)), ((), ()))
    s1_ref[...] += lax.dot_general(oh, h, dn, preferred_element_type=jnp.float32)
    s2_ref[...] += lax.dot_general(oh, h * h, dn,
                                   preferred_element_type=jnp.float32)
    cnt_ref[...] += lax.dot_general(oh, jnp.ones((_BN, 128), jnp.float32), dn,
                                    preferred_element_type=jnp.float32)


def _make_cheb1():
    return pl.pallas_call(
        _cheb1_body,
        grid=(_NB,),
        in_specs=[
            pl.BlockSpec((_BN, 128), lambda i: (i, 0)),
            pl.BlockSpec((_BN, 128), lambda i: (i, 0)),
            pl.BlockSpec((_BN, 128), lambda i: (i, 0)),
            pl.BlockSpec((_BN, 128), lambda i: (i, 0)),
            pl.BlockSpec((3, 128, 256), lambda i: (0, 0, 0)),
            pl.BlockSpec((1, 256), lambda i: (0, 0)),
            pl.BlockSpec((_BN, _G), lambda i: (i, 0)),
        ],
        out_specs=[
            pl.BlockSpec((_BN, 256), lambda i: (i, 0)),
            pl.BlockSpec((_G, 256), lambda i: (0, 0)),
            pl.BlockSpec((_G, 256), lambda i: (0, 0)),
            pl.BlockSpec((_G, 128), lambda i: (0, 0)),
        ],
        out_shape=[
            jax.ShapeDtypeStruct((_N, 256), jnp.float32),
            jax.ShapeDtypeStruct((_G, 256), jnp.float32),
            jax.ShapeDtypeStruct((_G, 256), jnp.float32),
            jax.ShapeDtypeStruct((_G, 128), jnp.float32),
        ],
    )


def _gn_body(h_ref, oh_ref, s1_ref, s2_ref, cnt_ref, gw_ref, gb_ref, gms_ref,
             ya_ref, yb_ref):
    cnt = jnp.maximum(cnt_ref[...][:, 0:1], 1.0)
    mean = s1_ref[...] / cnt
    msq = s2_ref[...] / cnt
    ms = gms_ref[...]
    var = msq - mean * mean * (ms * (2.0 - ms))
    rstd = lax.rsqrt(var + _EPS)
    oh = oh_ref[...]
    meanb = jnp.dot(oh, mean * ms, preferred_element_type=jnp.float32)
    rstdb = jnp.dot(oh, rstd, preferred_element_type=jnp.float32)
    y = jnp.maximum((h_ref[...] - meanb) * rstdb * gw_ref[...] + gb_ref[...],
                    0.0)
    ya_ref[...] = y[:, :128]
    yb_ref[...] = y[:, 128:]


def _make_gn():
    return pl.pallas_call(
        _gn_body,
        grid=(_NB,),
        in_specs=[
            pl.BlockSpec((_BN, 256), lambda i: (i, 0)),
            pl.BlockSpec((_BN, _G), lambda i: (i, 0)),
            pl.BlockSpec((_G, 256), lambda i: (0, 0)),
            pl.BlockSpec((_G, 256), lambda i: (0, 0)),
            pl.BlockSpec((_G, 128), lambda i: (0, 0)),
            pl.BlockSpec((1, 256), lambda i: (0, 0)),
            pl.BlockSpec((1, 256), lambda i: (0, 0)),
            pl.BlockSpec((1, 256), lambda i: (0, 0)),
        ],
        out_specs=[
            pl.BlockSpec((_BN, 128), lambda i: (i, 0)),
            pl.BlockSpec((_BN, 128), lambda i: (i, 0)),
        ],
        out_shape=[
            jax.ShapeDtypeStruct((_N, 128), jnp.float32),
            jax.ShapeDtypeStruct((_N, 128), jnp.float32),
        ],
    )


def _lin_body(pa_ref, pb_ref, w_ref, b_ref, out_ref):
    p = jnp.concatenate([pa_ref[...], pb_ref[...]], axis=1)
    p = jnp.where(jnp.isfinite(p), p, 0.0)
    out_ref[...] = jnp.dot(p, w_ref[...],
                           preferred_element_type=jnp.float32) + b_ref[...]


def _make_lin():
    return pl.pallas_call(
        _lin_body,
        out_shape=jax.ShapeDtypeStruct((_G, 128), jnp.float32),
    )


def _pad_nodes(a):
    return jnp.pad(a, ((0, _NP - _N), (0, 0)))


def kernel(x, edge_index, edge_weight, batch,
           W1, b1, gn1_w, gn1_b, gn1_ms,
           W2, b2, gn2_w, gn2_b, gn2_ms,
           W3, b3, gn3_w, gn3_b, gn3_ms,
           lin_w, lin_b):
    npad = _EP - _E
    pad_idx = (jnp.arange(npad, dtype=jnp.int32) * 7) % _N
    src_p = jnp.concatenate([edge_index[0], pad_idx])
    dst_p = jnp.concatenate([edge_index[1], pad_idx])
    w_p = jnp.concatenate([edge_weight, jnp.zeros((npad,), jnp.float32)])

    src16 = src_p.reshape(_NT, _DR, _EK)
    dst16 = dst_p.reshape(_NT, _DR, _EK)
    w16 = w_p.reshape(_NT, _DR, _EK)

    norm16 = _make_norm_kernel()(src16, dst16, w16)

    oh = (batch[:, None] == jnp.arange(_G, dtype=batch.dtype)[None, :]
          ).astype(jnp.float32)
    batch_p = jnp.concatenate(
        [batch, jnp.full((_NP - _N,), _G - 1, jnp.int32)]).reshape(_NT, _RT)

    prop = _make_prop(128)
    prop_es = _make_prop_es()
    addk = _make_add()
    cheb1 = _make_cheb1()
    cheb256 = _make_cheb(256)
    gn = _make_gn()

    def layer(xa, xb, W, b, gw, gb, gms):
        xap = _pad_nodes(xa)
        xbp = _pad_nodes(xb)
        t1a3, t1b3 = prop(xap, xbp, src16, dst16, norm16)
        t1ap = t1a3.reshape(_NP, 128)
        t1bp = t1b3.reshape(_NP, 128)
        t2a3, t2b3 = prop(t1ap, t1bp, src16, dst16, norm16)
        h, s1, s2, cnt = cheb256(xa, xb, t1ap, t1bp,
                                 t2a3.reshape(_NP, 128),
                                 t2b3.reshape(_NP, 128), W,
                                 b.reshape(1, 256), oh)
        ya, yb = gn(h, oh, s1, s2, cnt, gw.reshape(1, 256),
                    gb.reshape(1, 256), gms.reshape(1, 256))
        return ya, yb

    xp = _pad_nodes(x)
    p03, p13 = prop_es(xp, src16, dst16, norm16)
    t1p = addk(p03.reshape(_NP, 128), p13.reshape(_NP, 128))
    q03, q13 = prop_es(t1p, src16, dst16, norm16)
    h1, s11, s21, cnt1 = cheb1(x, t1p, q03.reshape(_NP, 128),
                               q13.reshape(_NP, 128), W1,
                               b1.reshape(1, 256), oh)
    y1a, y1b = gn(h1, oh, s11, s21, cnt1, gn1_w.reshape(1, 256),
                  gn1_b.reshape(1, 256), gn1_ms.reshape(1, 256))

    y2a, y2b = layer(y1a, y1b, W2, b2, gn2_w, gn2_b, gn2_ms)
    y3a, y3b = layer(y2a, y2b, W3, b3, gn3_w, gn3_b, gn3_ms)

    pa3, pb3 = _make_pool_kernel()(
        _pad_nodes(y3a).reshape(_NT, _RT, 128),
        _pad_nodes(y3b).reshape(_NT, _RT, 128), batch_p)

    lw = jnp.pad(lin_w, ((0, 0), (0, 112)))
    lb = jnp.pad(lin_b, (0, 112)).reshape(1, 128)
    out = _make_lin()(pa3.reshape(_G, 128), pb3.reshape(_G, 128), lw, lb)
    return out[:, :16]


# speedup vs baseline: 1.0326x; 1.0027x over previous
import functools

import jax
import jax.numpy as jnp
from jax import lax
from jax.experimental import pallas as pl
from jax.experimental.pallas import tpu as pltpu
from jax.experimental.pallas import tpu_sc as plsc

_N = 10000
_E = 320000
_G = 64
_EPS = 1e-5

_EK = 128
_EP = 327680
_ER = _EP // _EK
_NT = 16
_NC = 2
_DR = _ER // _NT
_NR = _ER // (_NT * _NC)
_NP = 10240
_RT = _NP // _NT
_BN = 400
_NB = _N // _BN


def _mesh():
    return plsc.VectorSubcoreMesh(core_axis_name="c", subcore_axis_name="s")


def _rsqrt_nr(x):
    i = lax.bitcast_convert_type(x, jnp.int32)
    i = jnp.int32(0x5F3759DF) - lax.shift_right_arithmetic(i, 1)
    y = lax.bitcast_convert_type(i, jnp.float32)
    for _ in range(4):
        y = y * (1.5 - 0.5 * x * y * y)
    return y


def _norm_kernel_fn(src16, dst16, w16, norm_out,
                    sidx, wval, degv, disv, nsrc, ndst, nw, nout, sdeg):
    c = lax.axis_index("c")
    s = lax.axis_index("s")

    @pl.when(s == 0)
    def _():
        def zrow(k, _):
            disv[pl.ds(k * 16, 16)] = jnp.zeros((16,), jnp.float32)
            return 0
        lax.fori_loop(0, _N // 16, zrow, 0)
        pltpu.sync_copy(disv, sdeg)

    plsc.subcore_barrier()

    pltpu.sync_copy(src16.at[s], sidx)
    pltpu.sync_copy(w16.at[s], wval)

    def degbody(j, _):
        pltpu.sync_copy(wval.at[j], sdeg.at[sidx.at[j]], add=True)
        return 0
    lax.fori_loop(0, _DR, degbody, 0)

    plsc.subcore_barrier()

    pltpu.sync_copy(sdeg, degv)

    def disbody(k, _):
        d = degv[pl.ds(k * 16, 16)]
        y = _rsqrt_nr(jnp.maximum(d, 1e-12))
        disv[pl.ds(k * 16, 16)] = jnp.where(d > 0.0, y, 0.0)
        return 0
    lax.fori_loop(0, _N // 16, disbody, 0)

    r0 = pl.multiple_of(c * _NR, _NR)
    pltpu.sync_copy(src16.at[s, pl.ds(r0, _NR)], nsrc)
    pltpu.sync_copy(dst16.at[s, pl.ds(r0, _NR)], ndst)
    pltpu.sync_copy(w16.at[s, pl.ds(r0, _NR)], nw)

    def nbody(j, _):
        for k in range(_EK // 16):
            sl = pl.ds(k * 16, 16)
            a = plsc.load_gather(disv, [nsrc[j, sl]])
            b = plsc.load_gather(disv, [ndst[j, sl]])
            nout[j, sl] = -(a * nw[j, sl] * b)
        return 0
    lax.fori_loop(0, _NR, nbody, 0)

    pltpu.sync_copy(nout, norm_out.at[s, pl.ds(r0, _NR)])


def _make_norm_kernel():
    return functools.partial(
        pl.kernel,
        out_type=jax.ShapeDtypeStruct((_NT, _DR, _EK), jnp.float32),
        mesh=_mesh(),
        compiler_params=pltpu.CompilerParams(needs_layout_passes=False),
        scratch_types=[
            pltpu.VMEM((_DR, _EK), jnp.int32),
            pltpu.VMEM((_DR, _EK), jnp.float32),
            pltpu.VMEM((_N,), jnp.float32),
            pltpu.VMEM((_N,), jnp.float32),
            pltpu.VMEM((_NR, _EK), jnp.int32),
            pltpu.VMEM((_NR, _EK), jnp.int32),
            pltpu.VMEM((_NR, _EK), jnp.float32),
            pltpu.VMEM((_NR, _EK), jnp.float32),
            pltpu.VMEM_SHARED((_N,), jnp.float32),
        ],
    )(_norm_kernel_fn)


def _make_prop(c2):
    def prop_fn(xa, xb, src16, dst16, norm16, ya3, yb3,
                sbuf, dbuf, nbuf, gbuf0, gbuf1, zbuf, acc, sem):
        c = lax.axis_index("c")
        s = lax.axis_index("s")

        def zrow(r, _):
            for v in range(c2 // 16):
                zbuf[r, pl.ds(v * 16, 16)] = jnp.zeros((16,), jnp.float32)
            return 0
        lax.fori_loop(0, 16, zrow, 0)

        def zcopy(k, _):
            r0 = pl.multiple_of(s * _RT + k * 16, 16)
            pltpu.sync_copy(zbuf, acc.at[pl.ds(r0, 16)])
            return 0
        lax.fori_loop(0, _RT // 16, zcopy, 0)

        plsc.subcore_barrier()

        def run(x_ref):
            def gstart(j, gb):
                pltpu.make_async_copy(x_ref.at[sbuf.at[j]], gb, sem).start()

            def gwait(j, gb):
                pltpu.make_async_copy(x_ref.at[sbuf.at[j]], gb, sem).wait()

            def scale(j, gb):
                def scale16(g, _):
                    nv16 = nbuf[j, pl.ds(g * 16, 16)]
                    for l in range(16):
                        e = g * 16 + l
                        nv = jnp.full((16,), nv16[l], jnp.float32)
                        for v in range(c2 // 16):
                            sl = pl.ds(v * 16, 16)
                            gb[e, sl] = gb[e, sl] * nv
                    return 0
                lax.fori_loop(0, _EK // 16, scale16, 0)

            def super_chunk(sc_i, _):
                r0 = pl.multiple_of(sc_i * 32, 32)
                pltpu.sync_copy(src16.at[s, pl.ds(r0, 32)], sbuf)
                pltpu.sync_copy(dst16.at[s, pl.ds(r0, 32)], dbuf)
                pltpu.sync_copy(norm16.at[s, pl.ds(r0, 32)], nbuf)
                gstart(0, gbuf0)

                def pair(k, _):
                    j0 = 2 * k
                    j1 = j0 + 1
                    gwait(j0, gbuf0)
                    gstart(j1, gbuf1)
                    scale(j0, gbuf0)
                    pltpu.sync_copy(gbuf0, acc.at[dbuf.at[j0]], add=True)
                    gwait(j1, gbuf1)

                    @pl.when(k < 15)
                    def _():
                        gstart(j0 + 2, gbuf0)
                    scale(j1, gbuf1)
                    pltpu.sync_copy(gbuf1, acc.at[dbuf.at[j1]], add=True)
                    return 0
                lax.fori_loop(0, 16, pair, 0)
                return 0
            lax.fori_loop(0, _DR // 32, super_chunk, 0)

        @pl.when(c == 0)
        def _():
            run(xa)

        @pl.when(c == 1)
        def _():
            run(xb)

        plsc.subcore_barrier()

        @pl.when(c == 0)
        def _():
            pltpu.sync_copy(acc.at[pl.ds(s * _RT, _RT)], ya3.at[s])

        @pl.when(c == 1)
        def _():
            pltpu.sync_copy(acc.at[pl.ds(s * _RT, _RT)], yb3.at[s])

    return functools.partial(
        pl.kernel,
        out_type=(jax.ShapeDtypeStruct((_NT, _RT, c2), jnp.float32),
                  jax.ShapeDtypeStruct((_NT, _RT, c2), jnp.float32)),
        mesh=_mesh(),
        compiler_params=pltpu.CompilerParams(needs_layout_passes=False),
        scratch_types=[
            pltpu.VMEM((32, _EK), jnp.int32),
            pltpu.VMEM((32, _EK), jnp.int32),
            pltpu.VMEM((32, _EK), jnp.float32),
            pltpu.VMEM((_EK, c2), jnp.float32),
            pltpu.VMEM((_EK, c2), jnp.float32),
            pltpu.VMEM((16, c2), jnp.float32),
            pltpu.VMEM_SHARED((_NP, c2), jnp.float32),
            pltpu.SemaphoreType.DMA,
        ],
    )(prop_fn)


def _make_prop_es():
    c2 = 128

    def prop_fn(x, src16, dst16, norm16, p03, p13,
                sbuf, dbuf, nbuf, gbuf0, gbuf1, zbuf, acc, sem):
        c = lax.axis_index("c")
        s = lax.axis_index("s")

        def zrow(r, _):
            for v in range(c2 // 16):
                zbuf[r, pl.ds(v * 16, 16)] = jnp.zeros((16,), jnp.float32)
            return 0
        lax.fori_loop(0, 32, zrow, 0)

        def zcopy(k, _):
            r0 = pl.multiple_of(s * _RT + k * 32, 32)
            pltpu.sync_copy(zbuf, acc.at[pl.ds(r0, 32)])
            return 0
        lax.fori_loop(0, _RT // 32, zcopy, 0)

        plsc.subcore_barrier()

        cbase = c * (_DR // 2)

        def gstart(j, gb):
            pltpu.make_async_copy(x.at[sbuf.at[j]], gb, sem).start()

        def gwait(j, gb):
            pltpu.make_async_copy(x.at[sbuf.at[j]], gb, sem).wait()

        def scale(j, gb):
            def scale16(g, _):
                nv16 = nbuf[j, pl.ds(g * 16, 16)]
                for l in range(16):
                    e = g * 16 + l
                    nv = jnp.full((16,), nv16[l], jnp.float32)
                    for v in range(c2 // 16):
                        sl = pl.ds(v * 16, 16)
                        gb[e, sl] = gb[e, sl] * nv
                return 0
            lax.fori_loop(0, _EK // 16, scale16, 0)

        def super_chunk(sc_i, _):
            r0 = pl.multiple_of(cbase + sc_i * 16, 16)
            pltpu.sync_copy(src16.at[s, pl.ds(r0, 16)], sbuf)
            pltpu.sync_copy(dst16.at[s, pl.ds(r0, 16)], dbuf)
            pltpu.sync_copy(norm16.at[s, pl.ds(r0, 16)], nbuf)
            gstart(0, gbuf0)

            def pair(k, _):
                j0 = 2 * k
                j1 = j0 + 1
                gwait(j0, gbuf0)
                gstart(j1, gbuf1)
                scale(j0, gbuf0)
                pltpu.sync_copy(gbuf0, acc.at[dbuf.at[j0]], add=True)
                gwait(j1, gbuf1)

                @pl.when(k < 7)
                def _():
                    gstart(j0 + 2, gbuf0)
                scale(j1, gbuf1)
                pltpu.sync_copy(gbuf1, acc.at[dbuf.at[j1]], add=True)
                return 0
            lax.fori_loop(0, 8, pair, 0)
            return 0
        lax.fori_loop(0, _DR // 2 // 16, super_chunk, 0)

        plsc.subcore_barrier()

        @pl.when(c == 0)
        def _():
            pltpu.sync_copy(acc.at[pl.ds(s * _RT, _RT)], p03.at[s])

        @pl.when(c == 1)
        def _():
            pltpu.sync_copy(acc.at[pl.ds(s * _RT, _RT)], p13.at[s])

    return functools.partial(
        pl.kernel,
        out_type=(jax.ShapeDtypeStruct((_NT, _RT, c2), jnp.float32),
                  jax.ShapeDtypeStruct((_NT, _RT, c2), jnp.float32)),
        mesh=_mesh(),
        compiler_params=pltpu.CompilerParams(needs_layout_passes=False),
        scratch_types=[
            pltpu.VMEM((16, _EK), jnp.int32),
            pltpu.VMEM((16, _EK), jnp.int32),
            pltpu.VMEM((16, _EK), jnp.float32),
            pltpu.VMEM((_EK, c2), jnp.float32),
            pltpu.VMEM((_EK, c2), jnp.float32),
            pltpu.VMEM((32, c2), jnp.float32),
            pltpu.VMEM_SHARED((_NP, c2), jnp.float32),
            pltpu.SemaphoreType.DMA,
        ],
    )(prop_fn)


def _add_body(a_ref, b_ref, o_ref):
    o_ref[...] = a_ref[...] + b_ref[...]


def _make_add():
    return pl.pallas_call(
        _add_body,
        grid=(_NT,),
        in_specs=[
            pl.BlockSpec((_RT, 128), lambda i: (i, 0)),
            pl.BlockSpec((_RT, 128), lambda i: (i, 0)),
        ],
        out_specs=pl.BlockSpec((_RT, 128), lambda i: (i, 0)),
        out_shape=jax.ShapeDtypeStruct((_NP, 128), jnp.float32),
    )


def _pool_kernel_fn(ya, yb, batch2d, pa3, pb3, rows, bidx, macc, tmp,
                    spacc):
    c = lax.axis_index("c")
    s = lax.axis_index("s")
    gpt = _G // _NT

    def irow(r, _):
        for v in range(8):
            macc[r, pl.ds(v * 16, 16)] = jnp.full((16,), -jnp.inf, jnp.float32)
        return 0
    lax.fori_loop(0, _G, irow, 0)

    pltpu.sync_copy(batch2d.at[s], bidx.at[pl.ds(0, _RT)])

    @pl.when(c == 0)
    def _():
        pltpu.sync_copy(ya.at[s], rows)

    @pl.when(c == 1)
    def _():
        pltpu.sync_copy(yb.at[s], rows)

    def rowbody(r, _):
        g = bidx[pl.ds(r, 16)][0]
        for v in range(8):
            sl = pl.ds(v * 16, 16)
            macc[g, sl] = jnp.maximum(macc[g, sl], rows[r, sl])
        return 0
    lax.fori_loop(0, _RT, rowbody, 0)

    pltpu.sync_copy(macc, spacc.at[s])
    plsc.subcore_barrier()

    def tbody(k, _):
        pltpu.sync_copy(spacc.at[k], tmp)

        def grow(r, _):
            for v in range(8):
                sl = pl.ds(v * 16, 16)
                macc[r, sl] = jnp.maximum(macc[r, sl], tmp[r, sl])
            return 0
        lax.fori_loop(0, _G, grow, 0)
        return 0
    lax.fori_loop(0, _NT, tbody, 0)

    @pl.when(c == 0)
    def _():
        pltpu.sync_copy(macc.at[pl.ds(s * gpt, gpt)], pa3.at[s])

    @pl.when(c == 1)
    def _():
        pltpu.sync_copy(macc.at[pl.ds(s * gpt, gpt)], pb3.at[s])


def _make_pool_kernel():
    gpt = _G // _NT
    return functools.partial(
        pl.kernel,
        out_type=(jax.ShapeDtypeStruct((_NT, gpt, 128), jnp.float32),
                  jax.ShapeDtypeStruct((_NT, gpt, 128), jnp.float32)),
        mesh=_mesh(),
        compiler_params=pltpu.CompilerParams(needs_layout_passes=False),
        scratch_types=[
            pltpu.VMEM((_RT, 128), jnp.float32),
            pltpu.VMEM((_RT + 16,), jnp.int32),
            pltpu.VMEM((_G, 128), jnp.float32),
            pltpu.VMEM((_G, 128), jnp.float32),
            pltpu.VMEM_SHARED((_NT, _G, 128), jnp.float32),
        ],
    )(_pool_kernel_fn)


def _cheb_body(h0_ref, t1a, t1b, t2a, t2b, w_ref, oh_ref,
               h_ref, s1_ref, s2_ref, cnt_ref):
    i = pl.program_id(0)
    t1 = jnp.concatenate([t1a[...], t1b[...]], axis=1)
    t2 = jnp.concatenate([t2a[...], t2b[...]], axis=1)
    a1 = w_ref[1]
    a2 = 2.0 * w_ref[2]
    h = (h0_ref[...]
         + jnp.dot(t1, a1, preferred_element_type=jnp.float32)
         + jnp.dot(t2, a2, preferred_element_type=jnp.float32))
    h_ref[...] = h
    oh = oh_ref[...]

    @pl.when(i == 0)
    def _():
        s1_ref[...] = jnp.zeros_like(s1_ref)
        s2_ref[...] = jnp.zeros_like(s2_ref)
        cnt_ref[...] = jnp.zeros_like(cnt_ref)

    dn = (((0,), (0,)), ((), ()))
    s1_ref[...] += lax.dot_general(oh, h, dn, preferred_element_type=jnp.float32)
    s2_ref[...] += lax.dot_general(oh, h * h, dn,
                                   preferred_element_type=jnp.float32)
    cnt_ref[...] += lax.dot_general(oh, jnp.ones((_BN, 128), jnp.float32), dn,
                                    preferred_element_type=jnp.float32)


def _make_cheb(cin):
    c2 = cin // 2
    return pl.pallas_call(
        _cheb_body,
        grid=(_NB,),
        in_specs=[
            pl.BlockSpec((_BN, 256), lambda i: (i, 0)),
            pl.BlockSpec((_BN, c2), lambda i: (i, 0)),
            pl.BlockSpec((_BN, c2), lambda i: (i, 0)),
            pl.BlockSpec((_BN, c2), lambda i: (i, 0)),
            pl.BlockSpec((_BN, c2), lambda i: (i, 0)),
            pl.BlockSpec((3, cin, 256), lambda i: (0, 0, 0)),
            pl.BlockSpec((_BN, _G), lambda i: (i, 0)),
        ],
        out_specs=[
            pl.BlockSpec((_BN, 256), lambda i: (i, 0)),
            pl.BlockSpec((_G, 256), lambda i: (0, 0)),
            pl.BlockSpec((_G, 256), lambda i: (0, 0)),
            pl.BlockSpec((_G, 128), lambda i: (0, 0)),
        ],
        out_shape=[
            jax.ShapeDtypeStruct((_N, 256), jnp.float32),
            jax.ShapeDtypeStruct((_G, 256), jnp.float32),
            jax.ShapeDtypeStruct((_G, 256), jnp.float32),
            jax.ShapeDtypeStruct((_G, 128), jnp.float32),
        ],
    )


def _cheb1_body(h0_ref, t1_ref, q0_ref, q1_ref, w_ref, oh_ref,
                h_ref, s1_ref, s2_ref, cnt_ref):
    i = pl.program_id(0)
    t1 = t1_ref[...]
    t2 = q0_ref[...] + q1_ref[...]
    a1 = w_ref[1]
    a2 = 2.0 * w_ref[2]
    h = (h0_ref[...]
         + jnp.dot(t1, a1, preferred_element_type=jnp.float32)
         + jnp.dot(t2, a2, preferred_element_type=jnp.float32))
    h_ref[...] = h
    oh = oh_ref[...]

    @pl.when(i == 0)
    def _():
        s1_ref[...] = jnp.zeros_like(s1_ref)
        s2_ref[...] = jnp.zeros_like(s2_ref)
        cnt_ref[...] = jnp.zeros_like(cnt_ref)

    dn = (((0,), (0,)), ((), ()))
    s1_ref[...] += lax.dot_general(oh, h, dn, preferred_element_type=jnp.float32)
    s2_ref[...] += lax.dot_general(oh, h * h, dn,
                                   preferred_element_type=jnp.float32)
    cnt_ref[...] += lax.dot_general(oh, jnp.ones((_BN, 128), jnp.float32), dn,
                                    preferred_element_type=jnp.float32)


def _make_cheb1():
    return pl.pallas_call(
        _cheb1_body,
        grid=(_NB,),
        in_specs=[
            pl.BlockSpec((_BN, 256), lambda i: (i, 0)),
            pl.BlockSpec((_BN, 128), lambda i: (i, 0)),
            pl.BlockSpec((_BN, 128), lambda i: (i, 0)),
            pl.BlockSpec((_BN, 128), lambda i: (i, 0)),
            pl.BlockSpec((3, 128, 256), lambda i: (0, 0, 0)),
            pl.BlockSpec((_BN, _G), lambda i: (i, 0)),
        ],
        out_specs=[
            pl.BlockSpec((_BN, 256), lambda i: (i, 0)),
            pl.BlockSpec((_G, 256), lambda i: (0, 0)),
            pl.BlockSpec((_G, 256), lambda i: (0, 0)),
            pl.BlockSpec((_G, 128), lambda i: (0, 0)),
        ],
        out_shape=[
            jax.ShapeDtypeStruct((_N, 256), jnp.float32),
            jax.ShapeDtypeStruct((_G, 256), jnp.float32),
            jax.ShapeDtypeStruct((_G, 256), jnp.float32),
            jax.ShapeDtypeStruct((_G, 128), jnp.float32),
        ],
    )


def _xpart2_body(xa, xb, w_ref, b_ref, h0_ref):
    x = jnp.concatenate([xa[...], xb[...]], axis=1)
    a0 = w_ref[0] - w_ref[2]
    h0_ref[...] = jnp.dot(x, a0, preferred_element_type=jnp.float32) + b_ref[...]


def _make_xpart2(cin):
    c2 = cin // 2
    return pl.pallas_call(
        _xpart2_body,
        grid=(_NB,),
        in_specs=[
            pl.BlockSpec((_BN, c2), lambda i: (i, 0)),
            pl.BlockSpec((_BN, c2), lambda i: (i, 0)),
            pl.BlockSpec((3, cin, 256), lambda i: (0, 0, 0)),
            pl.BlockSpec((1, 256), lambda i: (0, 0)),
        ],
        out_specs=pl.BlockSpec((_BN, 256), lambda i: (i, 0)),
        out_shape=jax.ShapeDtypeStruct((_N, 256), jnp.float32),
    )


def _xpart1_body(x_ref, w_ref, b_ref, h0_ref):
    a0 = w_ref[0] - w_ref[2]
    h0_ref[...] = (jnp.dot(x_ref[...], a0, preferred_element_type=jnp.float32)
                   + b_ref[...])


def _make_xpart1():
    return pl.pallas_call(
        _xpart1_body,
        grid=(_NB,),
        in_specs=[
            pl.BlockSpec((_BN, 128), lambda i: (i, 0)),
            pl.BlockSpec((3, 128, 256), lambda i: (0, 0, 0)),
            pl.BlockSpec((1, 256), lambda i: (0, 0)),
        ],
        out_specs=pl.BlockSpec((_BN, 256), lambda i: (i, 0)),
        out_shape=jax.ShapeDtypeStruct((_N, 256), jnp.float32),
    )


def _gn_body(h_ref, oh_ref, s1_ref, s2_ref, cnt_ref, gw_ref, gb_ref, gms_ref,
             ya_ref, yb_ref):
    cnt = jnp.maximum(cnt_ref[...][:, 0:1], 1.0)
    mean = s1_ref[...] / cnt
    msq = s2_ref[...] / cnt
    ms = gms_ref[...]
    var = msq - mean * mean * (ms * (2.0 - ms))
    rstd = lax.rsqrt(var + _EPS)
    oh = oh_ref[...]
    meanb = jnp.dot(oh, mean * ms, preferred_element_type=jnp.float32)
    rstdb = jnp.dot(oh, rstd, preferred_element_type=jnp.float32)
    y = jnp.maximum((h_ref[...] - meanb) * rstdb * gw_ref[...] + gb_ref[...],
                    0.0)
    ya_ref[...] = y[:, :128]
    yb_ref[...] = y[:, 128:]


def _make_gn():
    return pl.pallas_call(
        _gn_body,
        grid=(_NB,),
        in_specs=[
            pl.BlockSpec((_BN, 256), lambda i: (i, 0)),
            pl.BlockSpec((_BN, _G), lambda i: (i, 0)),
            pl.BlockSpec((_G, 256), lambda i: (0, 0)),
            pl.BlockSpec((_G, 256), lambda i: (0, 0)),
            pl.BlockSpec((_G, 128), lambda i: (0, 0)),
            pl.BlockSpec((1, 256), lambda i: (0, 0)),
            pl.BlockSpec((1, 256), lambda i: (0, 0)),
            pl.BlockSpec((1, 256), lambda i: (0, 0)),
        ],
        out_specs=[
            pl.BlockSpec((_BN, 128), lambda i: (i, 0)),
            pl.BlockSpec((_BN, 128), lambda i: (i, 0)),
        ],
        out_shape=[
            jax.ShapeDtypeStruct((_N, 128), jnp.float32),
            jax.ShapeDtypeStruct((_N, 128), jnp.float32),
        ],
    )


def _lin_body(pa_ref, pb_ref, w_ref, b_ref, out_ref):
    p = jnp.concatenate([pa_ref[...], pb_ref[...]], axis=1)
    p = jnp.where(jnp.isfinite(p), p, 0.0)
    out_ref[...] = jnp.dot(p, w_ref[...],
                           preferred_element_type=jnp.float32) + b_ref[...]


def _make_lin():
    return pl.pallas_call(
        _lin_body,
        out_shape=jax.ShapeDtypeStruct((_G, 128), jnp.float32),
    )


def _pad_nodes(a):
    return jnp.pad(a, ((0, _NP - _N), (0, 0)))


def kernel(x, edge_index, edge_weight, batch,
           W1, b1, gn1_w, gn1_b, gn1_ms,
           W2, b2, gn2_w, gn2_b, gn2_ms,
           W3, b3, gn3_w, gn3_b, gn3_ms,
           lin_w, lin_b):
    npad = _EP - _E
    pad_idx = (jnp.arange(npad, dtype=jnp.int32) * 7) % _N
    src_p = jnp.concatenate([edge_index[0], pad_idx])
    dst_p = jnp.concatenate([edge_index[1], pad_idx])
    w_p = jnp.concatenate([edge_weight, jnp.zeros((npad,), jnp.float32)])

    src16 = src_p.reshape(_NT, _DR, _EK)
    dst16 = dst_p.reshape(_NT, _DR, _EK)
    w16 = w_p.reshape(_NT, _DR, _EK)

    norm16 = _make_norm_kernel()(src16, dst16, w16)

    oh = (batch[:, None] == jnp.arange(_G, dtype=batch.dtype)[None, :]
          ).astype(jnp.float32)
    batch_p = jnp.concatenate(
        [batch, jnp.full((_NP - _N,), _G - 1, jnp.int32)]).reshape(_NT, _RT)

    prop = _make_prop(128)
    prop_es = _make_prop_es()
    addk = _make_add()
    cheb1 = _make_cheb1()
    cheb256 = _make_cheb(256)
    xpart1 = _make_xpart1()
    xpart2 = _make_xpart2(256)
    gn = _make_gn()

    def layer(xa, xb, W, b, gw, gb, gms):
        h0 = xpart2(xa, xb, W, b.reshape(1, 256))
        xap = _pad_nodes(xa)
        xbp = _pad_nodes(xb)
        t1a3, t1b3 = prop(xap, xbp, src16, dst16, norm16)
        t1ap = t1a3.reshape(_NP, 128)
        t1bp = t1b3.reshape(_NP, 128)
        t2a3, t2b3 = prop(t1ap, t1bp, src16, dst16, norm16)
        h, s1, s2, cnt = cheb256(h0, t1ap, t1bp,
                                 t2a3.reshape(_NP, 128),
                                 t2b3.reshape(_NP, 128), W, oh)
        ya, yb = gn(h, oh, s1, s2, cnt, gw.reshape(1, 256),
                    gb.reshape(1, 256), gms.reshape(1, 256))
        return ya, yb

    h01 = xpart1(x, W1, b1.reshape(1, 256))
    xp = _pad_nodes(x)
    p03, p13 = prop_es(xp, src16, dst16, norm16)
    t1p = addk(p03.reshape(_NP, 128), p13.reshape(_NP, 128))
    q03, q13 = prop_es(t1p, src16, dst16, norm16)
    h1, s11, s21, cnt1 = cheb1(h01, t1p, q03.reshape(_NP, 128),
                               q13.reshape(_NP, 128), W1, oh)
    y1a, y1b = gn(h1, oh, s11, s21, cnt1, gn1_w.reshape(1, 256),
                  gn1_b.reshape(1, 256), gn1_ms.reshape(1, 256))

    y2a, y2b = layer(y1a, y1b, W2, b2, gn2_w, gn2_b, gn2_ms)
    y3a, y3b = layer(y2a, y2b, W3, b3, gn3_w, gn3_b, gn3_ms)

    pa3, pb3 = _make_pool_kernel()(
        _pad_nodes(y3a).reshape(_NT, _RT, 128),
        _pad_nodes(y3b).reshape(_NT, _RT, 128), batch_p)

    lw = jnp.pad(lin_w, ((0, 0), (0, 112)))
    lb = jnp.pad(lin_b, (0, 112)).reshape(1, 128)
    out = _make_lin()(pa3.reshape(_G, 128), pb3.reshape(_G, 128), lw, lb)
    return out[:, :16]
